# Initial kernel scaffold; baseline (speedup 1.0000x reference)
import jax
import jax.numpy as jnp
from jax.experimental import pallas as pl


def kernel(edge_embeddings, edge_index, edge_attr, node_embeddings, num_nodes, W_attn, b_attn, W_update, b_update, W_edge, b_edge, W_node, b_node):
    raise NotImplementedError("write your pallas kernel here")


# speedup vs baseline: 1.7713x; 1.7713x over previous
import functools

import jax
import jax.numpy as jnp
from jax import lax
from jax.experimental import pallas as pl
from jax.experimental.pallas import tpu as pltpu
from jax.experimental.pallas import tpu_sc as plsc

_NC = 2
_NS = 16
_NW = _NC * _NS
_SUB = 80
_NSUB = 5
_CHUNK = _SUB * _NSUB

_BE = 4000


def _tables_body(node_ref, ws_ref, wd_ref, was_ref, wad_ref, bn_ref,
                 s_ref, d_ref, as_ref, ad_ref):
    x = node_ref[...]
    s_ref[...] = jnp.dot(x, ws_ref[...], preferred_element_type=jnp.float32) + bn_ref[...]
    d_ref[...] = jnp.dot(x, wd_ref[...], preferred_element_type=jnp.float32) + bn_ref[...]
    as_ref[...] = jnp.dot(x, was_ref[...], preferred_element_type=jnp.float32)
    ad_ref[...] = jnp.dot(x, wad_ref[...], preferred_element_type=jnp.float32)


def _sc_body(src_h, dst_h, s_hbm, d_hbm, as_hbm, ad_hbm,
             gu_hbm, gf_hbm, ga_hbm,
             idx_s, idx_d, s_rows, d_rows, as_rows, ad_rows,
             gu_v, gf_v, ga_v, sem,
             *, edges_per_worker):
    cid = lax.axis_index("c")
    sid = lax.axis_index("s")
    wid = sid * _NC + cid
    n_chunks = edges_per_worker // _CHUNK

    def chunk(c, carry):
        base = wid * edges_per_worker + c * _CHUNK
        pltpu.sync_copy(src_h.at[pl.ds(base, _CHUNK)], idx_s)
        pltpu.sync_copy(dst_h.at[pl.ds(base, _CHUNK)], idx_d)
        cps = []
        for k in range(_NSUB):
            i_s = idx_s.at[pl.ds(k * _SUB, _SUB)]
            i_d = idx_d.at[pl.ds(k * _SUB, _SUB)]
            cps.append(pltpu.async_copy(s_hbm.at[i_s], s_rows.at[k], sem))
            cps.append(pltpu.async_copy(d_hbm.at[i_d], d_rows.at[k], sem))
            cps.append(pltpu.async_copy(as_hbm.at[i_s], as_rows.at[k], sem))
            cps.append(pltpu.async_copy(ad_hbm.at[i_d], ad_rows.at[k], sem))
        for cp in cps:
            cp.wait()
        for k in range(_NSUB):
            def ga_body(j, c2, k=k):
                ga_v[pl.ds(k * _SUB + j * 16, 16)] = (
                    as_rows[k, pl.ds(j * 16, 16)] + ad_rows[k, pl.ds(j * 16, 16)])
                return c2
            lax.fori_loop(0, _SUB // 16, ga_body, 0)

            def add_body(e, c2, k=k):
                r = k * _SUB + e
                gu_v[r, :] = s_rows[k, e, pl.ds(0, 16)] + d_rows[k, e, pl.ds(0, 16)]
                gf_v[r, :] = s_rows[k, e, pl.ds(16, 16)] + d_rows[k, e, pl.ds(16, 16)]
                return c2
            lax.fori_loop(0, _SUB, add_body, 0)
        pltpu.sync_copy(gu_v, gu_hbm.at[pl.ds(base, _CHUNK)])
        pltpu.sync_copy(gf_v, gf_hbm.at[pl.ds(base, _CHUNK)])
        pltpu.sync_copy(ga_v, ga_hbm.at[pl.ds(base, _CHUNK)])
        return carry

    lax.fori_loop(0, n_chunks, chunk, 0)


def _mid_body(gu, gf, ga, emb, wue, wae, wedge, bu, ba, be, upd, stats):
    i = pl.program_id(0)
    e = emb[...]
    u_e = jnp.dot(e, wue[...], preferred_element_type=jnp.float32) + bu[...]
    a_e = jnp.dot(e, wae[...], preferred_element_type=jnp.float32) + ba[...]
    a = ga[...] + a_e
    a = jnp.maximum(a, 0.2 * a)
    attn = 1.0 / (1.0 + jnp.exp(-a))
    upd_blk = (gu[...] + u_e) * attn
    upd[...] = upd_blk
    ef = jnp.dot(upd_blk, wedge[...], preferred_element_type=jnp.float32) + gf[...] + be[...]
    s = jnp.maximum(ef, 0.2 * ef)
    m_blk = jnp.max(s, axis=0, keepdims=True)

    @pl.when(i == 0)
    def _():
        stats[0:1, :] = m_blk
        stats[1:2, :] = jnp.sum(jnp.exp(s - m_blk), axis=0, keepdims=True)

    @pl.when(i > 0)
    def _():
        m_old = stats[0:1, :]
        l_old = stats[1:2, :]
        m_new = jnp.maximum(m_old, m_blk)
        stats[0:1, :] = m_new
        stats[1:2, :] = (l_old * jnp.exp(m_old - m_new)
                         + jnp.sum(jnp.exp(s - m_new), axis=0, keepdims=True))


def _final_body(upd, gf, wedge, be, stats, out):
    u = upd[...]
    ef = jnp.dot(u, wedge[...], preferred_element_type=jnp.float32) + gf[...] + be[...]
    s = jnp.maximum(ef, 0.2 * ef)
    m = stats[0:1, :]
    l = stats[1:2, :]
    out[...] = u * (jnp.exp(s - m) * (1.0 / l))


def kernel(edge_embeddings, edge_index, edge_attr, node_embeddings, num_nodes,
           W_attn, b_attn, W_update, b_update, W_edge, b_edge, W_node, b_node):
    E, ED = edge_embeddings.shape
    N, ND = node_embeddings.shape
    OC = W_edge.shape[1]
    f32 = jnp.float32

    src = edge_index[0].astype(jnp.int32)
    dst = edge_index[1].astype(jnp.int32)

    wS = jnp.concatenate([W_update[:ND], W_node], axis=1)
    wD = jnp.concatenate([W_update[ND:2 * ND], W_node], axis=1)
    wAs = W_attn[:ND]
    wAd = W_attn[ND:2 * ND]
    wue = W_update[2 * ND:]
    wae = W_attn[2 * ND:]
    bn_row = jnp.concatenate([jnp.zeros((OC,), f32), b_node]).reshape(1, 2 * OC)

    S, D, As, Ad = pl.pallas_call(
        _tables_body,
        out_shape=[
            jax.ShapeDtypeStruct((N, 2 * OC), f32),
            jax.ShapeDtypeStruct((N, 2 * OC), f32),
            jax.ShapeDtypeStruct((N, 1), f32),
            jax.ShapeDtypeStruct((N, 1), f32),
        ],
    )(node_embeddings, wS, wD, wAs, wAd, bn_row)

    edges_per_worker = E // _NW

    mesh = plsc.VectorSubcoreMesh(
        core_axis_name="c", subcore_axis_name="s",
        num_cores=_NC, num_subcores=_NS)
    sc_gather = pl.kernel(
        functools.partial(_sc_body, edges_per_worker=edges_per_worker),
        out_type=[
            jax.ShapeDtypeStruct((E, OC), f32),
            jax.ShapeDtypeStruct((E, OC), f32),
            jax.ShapeDtypeStruct((E,), f32),
        ],
        mesh=mesh,
        scratch_types=[
            pltpu.VMEM((_CHUNK,), jnp.int32),
            pltpu.VMEM((_CHUNK,), jnp.int32),
            pltpu.VMEM((_NSUB, _SUB, 2 * OC), f32),
            pltpu.VMEM((_NSUB, _SUB, 2 * OC), f32),
            pltpu.VMEM((_NSUB, _SUB), f32),
            pltpu.VMEM((_NSUB, _SUB), f32),
            pltpu.VMEM((_CHUNK, OC), f32),
            pltpu.VMEM((_CHUNK, OC), f32),
            pltpu.VMEM((_CHUNK,), f32),
            pltpu.SemaphoreType.DMA,
        ],
        compiler_params=pltpu.CompilerParams(use_tc_tiling_on_sc=False),
    )
    Gu, Gf, Ga1 = sc_gather(src, dst, S, D, As.reshape(N), Ad.reshape(N))
    Ga = Ga1.reshape(E, 1)

    nb = E // _BE
    bspec_e = pl.BlockSpec((_BE, OC), lambda i: (i, 0))
    upd, stats = pl.pallas_call(
        _mid_body,
        grid=(nb,),
        in_specs=[
            bspec_e,
            bspec_e,
            pl.BlockSpec((_BE, 1), lambda i: (i, 0)),
            pl.BlockSpec((_BE, ED), lambda i: (i, 0)),
            pl.BlockSpec((ED, OC), lambda i: (0, 0)),
            pl.BlockSpec((ED, 1), lambda i: (0, 0)),
            pl.BlockSpec((OC, OC), lambda i: (0, 0)),
            pl.BlockSpec((1, OC), lambda i: (0, 0)),
            pl.BlockSpec((1, 1), lambda i: (0, 0)),
            pl.BlockSpec((1, OC), lambda i: (0, 0)),
        ],
        out_specs=[
            bspec_e,
            pl.BlockSpec((2, OC), lambda i: (0, 0)),
        ],
        out_shape=[
            jax.ShapeDtypeStruct((E, OC), f32),
            jax.ShapeDtypeStruct((2, OC), f32),
        ],
    )(Gu, Gf, Ga, edge_embeddings, wue, wae, W_edge,
      b_update.reshape(1, OC), b_attn.reshape(1, 1), b_edge.reshape(1, OC))

    refined = pl.pallas_call(
        _final_body,
        grid=(nb,),
        in_specs=[
            bspec_e,
            bspec_e,
            pl.BlockSpec((OC, OC), lambda i: (0, 0)),
            pl.BlockSpec((1, OC), lambda i: (0, 0)),
            pl.BlockSpec((2, OC), lambda i: (0, 0)),
        ],
        out_specs=bspec_e,
        out_shape=jax.ShapeDtypeStruct((E, OC), f32),
    )(upd, Gf, W_edge, b_edge.reshape(1, OC), stats)

    return refined


# speedup vs baseline: 3.2379x; 1.8280x over previous
import functools

import jax
import jax.numpy as jnp
from jax import lax
from jax.experimental import pallas as pl
from jax.experimental.pallas import tpu as pltpu
from jax.experimental.pallas import tpu_sc as plsc

_NC = 2
_NS = 16
_NW = _NC * _NS
_SUB = 80
_NSUB = 5
_CHUNK = _SUB * _NSUB

_BF = 1000


def _tables_body(node_ref, ws_ref, wd_ref, was_ref, wad_ref, bn_ref,
                 s_ref, d_ref, as_ref, ad_ref):
    x = node_ref[...]
    s_ref[...] = jnp.dot(x, ws_ref[...], preferred_element_type=jnp.float32) + bn_ref[...]
    d_ref[...] = jnp.dot(x, wd_ref[...], preferred_element_type=jnp.float32) + bn_ref[...]
    as_ref[...] = jnp.dot(x, was_ref[...], preferred_element_type=jnp.float32)
    ad_ref[...] = jnp.dot(x, wad_ref[...], preferred_element_type=jnp.float32)


def _sc_body(src_h, dst_h, s_hbm, d_hbm, as_hbm, ad_hbm,
             gu_hbm, gf_hbm, ga_hbm,
             idx_s, idx_d, s_rows, d_rows, as_rows, ad_rows,
             gu_v, gf_v, ga_v, sem,
             *, edges_per_worker):
    cid = lax.axis_index("c")
    sid = lax.axis_index("s")
    wid = sid * _NC + cid
    n_chunks = edges_per_worker // _CHUNK

    def chunk(c, carry):
        base = wid * edges_per_worker + c * _CHUNK
        pltpu.sync_copy(src_h.at[pl.ds(base, _CHUNK)], idx_s)
        pltpu.sync_copy(dst_h.at[pl.ds(base, _CHUNK)], idx_d)
        cps = []
        for k in range(_NSUB):
            i_s = idx_s.at[pl.ds(k * _SUB, _SUB)]
            i_d = idx_d.at[pl.ds(k * _SUB, _SUB)]
            cps.append(pltpu.async_copy(s_hbm.at[i_s], s_rows.at[k], sem))
            cps.append(pltpu.async_copy(d_hbm.at[i_d], d_rows.at[k], sem))
            cps.append(pltpu.async_copy(as_hbm.at[i_s], as_rows.at[k], sem))
            cps.append(pltpu.async_copy(ad_hbm.at[i_d], ad_rows.at[k], sem))
        for cp in cps:
            cp.wait()
        for k in range(_NSUB):
            def ga_body(j, c2, k=k):
                ga_v[pl.ds(k * _SUB + j * 16, 16)] = (
                    as_rows[k, pl.ds(j * 16, 16)] + ad_rows[k, pl.ds(j * 16, 16)])
                return c2
            lax.fori_loop(0, _SUB // 16, ga_body, 0)

            def add_body(e, c2, k=k):
                r = k * _SUB + e
                fr = r // 8
                sl = (r % 8) * 16
                gu_v[fr, pl.ds(sl, 16)] = s_rows[k, e, pl.ds(0, 16)] + d_rows[k, e, pl.ds(0, 16)]
                gf_v[fr, pl.ds(sl, 16)] = s_rows[k, e, pl.ds(16, 16)] + d_rows[k, e, pl.ds(16, 16)]
                return c2
            lax.fori_loop(0, _SUB, add_body, 0)
        pltpu.sync_copy(gu_v, gu_hbm.at[pl.ds(base // 8, _CHUNK // 8)])
        pltpu.sync_copy(gf_v, gf_hbm.at[pl.ds(base // 8, _CHUNK // 8)])
        pltpu.sync_copy(ga_v, ga_hbm.at[pl.ds(base, _CHUNK)])
        return carry

    lax.fori_loop(0, n_chunks, chunk, 0)


def _mid_body(gu, gf, ga, emb, wue, wae, wedge, bu, ba, be, upd, stats):
    i = pl.program_id(0)
    e = emb[...]
    u_e = jnp.dot(e, wue[...], preferred_element_type=jnp.float32) + bu[...]
    a_e = jnp.dot(e, wae[...], preferred_element_type=jnp.float32) + ba[...]
    z = ga[...] + a_e
    z = jnp.maximum(z, 0.2 * z)
    attn = 1.0 / (1.0 + jnp.exp(-z))
    upd_blk = (gu[...] + u_e) * attn
    upd[...] = upd_blk
    ef = jnp.dot(upd_blk, wedge[...], preferred_element_type=jnp.float32) + gf[...] + be[...]
    s = jnp.maximum(ef, 0.2 * ef)
    m_blk = jnp.max(s, axis=0, keepdims=True)

    @pl.when(i == 0)
    def _():
        stats[0:1, :] = m_blk
        stats[1:2, :] = jnp.sum(jnp.exp(s - m_blk), axis=0, keepdims=True)

    @pl.when(i > 0)
    def _():
        m_old = stats[0:1, :]
        l_old = stats[1:2, :]
        m_new = jnp.maximum(m_old, m_blk)
        stats[0:1, :] = m_new
        stats[1:2, :] = (l_old * jnp.exp(m_old - m_new)
                         + jnp.sum(jnp.exp(s - m_new), axis=0, keepdims=True))


def _final_body(upd, gf, wedge, be, stats, out, *, oc):
    u = upd[...]
    ef = jnp.dot(u, wedge[...], preferred_element_type=jnp.float32) + gf[...] + be[...]
    s = jnp.maximum(ef, 0.2 * ef)
    ms = stats[0:1, :]
    ls = stats[1:2, :]
    m_col = ms[:, 0:oc]
    for a in range(1, 8):
        m_col = jnp.maximum(m_col, ms[:, a * oc:(a + 1) * oc])
    l_col = jnp.zeros_like(m_col)
    for a in range(8):
        l_col = l_col + ls[:, a * oc:(a + 1) * oc] * jnp.exp(ms[:, a * oc:(a + 1) * oc] - m_col)
    m_t = jnp.concatenate([m_col] * 8, axis=1)
    l_t = jnp.concatenate([l_col] * 8, axis=1)
    out[...] = u * (jnp.exp(s - m_t) * (1.0 / l_t))


def kernel(edge_embeddings, edge_index, edge_attr, node_embeddings, num_nodes,
           W_attn, b_attn, W_update, b_update, W_edge, b_edge, W_node, b_node):
    E, ED = edge_embeddings.shape
    N, ND = node_embeddings.shape
    OC = W_edge.shape[1]
    f32 = jnp.float32

    src = edge_index[0].astype(jnp.int32)
    dst = edge_index[1].astype(jnp.int32)

    wS = jnp.concatenate([W_update[:ND], W_node], axis=1)
    wD = jnp.concatenate([W_update[ND:2 * ND], W_node], axis=1)
    wAs = W_attn[:ND]
    wAd = W_attn[ND:2 * ND]
    wue = W_update[2 * ND:]
    wae = W_attn[2 * ND:]
    bn_row = jnp.concatenate([jnp.zeros((OC,), f32), b_node]).reshape(1, 2 * OC)

    S, D, As, Ad = pl.pallas_call(
        _tables_body,
        out_shape=[
            jax.ShapeDtypeStruct((N, 2 * OC), f32),
            jax.ShapeDtypeStruct((N, 2 * OC), f32),
            jax.ShapeDtypeStruct((N, 1), f32),
            jax.ShapeDtypeStruct((N, 1), f32),
        ],
    )(node_embeddings, wS, wD, wAs, wAd, bn_row)

    edges_per_worker = E // _NW

    mesh = plsc.VectorSubcoreMesh(
        core_axis_name="c", subcore_axis_name="s",
        num_cores=_NC, num_subcores=_NS)
    sc_gather = pl.kernel(
        functools.partial(_sc_body, edges_per_worker=edges_per_worker),
        out_type=[
            jax.ShapeDtypeStruct((E // 8, 8 * OC), f32),
            jax.ShapeDtypeStruct((E // 8, 8 * OC), f32),
            jax.ShapeDtypeStruct((E,), f32),
        ],
        mesh=mesh,
        scratch_types=[
            pltpu.VMEM((_CHUNK,), jnp.int32),
            pltpu.VMEM((_CHUNK,), jnp.int32),
            pltpu.VMEM((_NSUB, _SUB, 2 * OC), f32),
            pltpu.VMEM((_NSUB, _SUB, 2 * OC), f32),
            pltpu.VMEM((_NSUB, _SUB), f32),
            pltpu.VMEM((_NSUB, _SUB), f32),
            pltpu.VMEM((_CHUNK // 8, 8 * OC), f32),
            pltpu.VMEM((_CHUNK // 8, 8 * OC), f32),
            pltpu.VMEM((_CHUNK,), f32),
            pltpu.SemaphoreType.DMA,
        ],
        compiler_params=pltpu.CompilerParams(use_tc_tiling_on_sc=False),
    )
    Gu, Gf, Ga1 = sc_gather(src, dst, S, D, As.reshape(N), Ad.reshape(N))
    Ga = jnp.repeat(Ga1, OC).reshape(E // 8, 8 * OC)
    emb_fold = edge_embeddings.reshape(E // 8, 8 * ED)

    eye8 = jnp.eye(8, dtype=f32)
    wue_k = jnp.kron(eye8, wue)
    wae_k = jnp.kron(eye8, wae @ jnp.ones((1, OC), f32))
    wedge_k = jnp.kron(eye8, W_edge)
    bu_t = jnp.tile(b_update.reshape(1, OC), (1, 8))
    ba_t = jnp.tile(b_attn.reshape(1, 1), (1, 8 * OC))
    be_t = jnp.tile(b_edge.reshape(1, OC), (1, 8))

    EF = E // 8
    nb = EF // _BF
    bspec_f = pl.BlockSpec((_BF, 8 * OC), lambda i: (i, 0))
    wspec = pl.BlockSpec((8 * OC, 8 * OC), lambda i: (0, 0))
    bias_spec = pl.BlockSpec((1, 8 * OC), lambda i: (0, 0))
    upd, stats = pl.pallas_call(
        _mid_body,
        grid=(nb,),
        in_specs=[bspec_f, bspec_f, bspec_f, bspec_f,
                  wspec, wspec, wspec, bias_spec, bias_spec, bias_spec],
        out_specs=[bspec_f, pl.BlockSpec((2, 8 * OC), lambda i: (0, 0))],
        out_shape=[
            jax.ShapeDtypeStruct((EF, 8 * OC), f32),
            jax.ShapeDtypeStruct((2, 8 * OC), f32),
        ],
    )(Gu, Gf, Ga, emb_fold, wue_k, wae_k, wedge_k, bu_t, ba_t, be_t)

    refined_fold = pl.pallas_call(
        functools.partial(_final_body, oc=OC),
        grid=(nb,),
        in_specs=[bspec_f, bspec_f, wspec, bias_spec,
                  pl.BlockSpec((2, 8 * OC), lambda i: (0, 0))],
        out_specs=bspec_f,
        out_shape=jax.ShapeDtypeStruct((EF, 8 * OC), f32),
    )(upd, Gf, wedge_k, be_t, stats)

    return refined_fold.reshape(E, OC)


# speedup vs baseline: 3.3849x; 1.0454x over previous
import functools

import jax
import jax.numpy as jnp
from jax import lax
from jax.experimental import pallas as pl
from jax.experimental.pallas import tpu as pltpu
from jax.experimental.pallas import tpu_sc as plsc

_NC = 2
_NS = 16
_NW = _NC * _NS
_SUB = 80
_NSUB = 5
_CHUNK = _SUB * _NSUB

_BF = 2000


def _tables_body(node_ref, ws_ref, wd_ref, was_ref, wad_ref, bn_ref,
                 s_ref, d_ref, as_ref, ad_ref):
    x = node_ref[...]
    s_ref[...] = jnp.dot(x, ws_ref[...], preferred_element_type=jnp.float32) + bn_ref[...]
    d_ref[...] = jnp.dot(x, wd_ref[...], preferred_element_type=jnp.float32) + bn_ref[...]
    as_ref[...] = jnp.dot(x, was_ref[...], preferred_element_type=jnp.float32)
    ad_ref[...] = jnp.dot(x, wad_ref[...], preferred_element_type=jnp.float32)


def _sc_body(src_h, dst_h, s_hbm, d_hbm, as_hbm, ad_hbm,
             gu_hbm, gf_hbm, ga_hbm,
             idx_s, idx_d, s_rows, d_rows, as_rows, ad_rows,
             gu_v, gf_v, ga_v, sem,
             *, edges_per_worker):
    cid = lax.axis_index("c")
    sid = lax.axis_index("s")
    wid = sid * _NC + cid
    n_chunks = edges_per_worker // _CHUNK

    def chunk(c, carry):
        base = wid * edges_per_worker + c * _CHUNK
        pltpu.sync_copy(src_h.at[pl.ds(base, _CHUNK)], idx_s)
        pltpu.sync_copy(dst_h.at[pl.ds(base, _CHUNK)], idx_d)
        cps = []
        for k in range(_NSUB):
            i_s = idx_s.at[pl.ds(k * _SUB, _SUB)]
            i_d = idx_d.at[pl.ds(k * _SUB, _SUB)]
            cps.append(pltpu.async_copy(s_hbm.at[i_s], s_rows.at[k], sem))
            cps.append(pltpu.async_copy(d_hbm.at[i_d], d_rows.at[k], sem))
            cps.append(pltpu.async_copy(as_hbm.at[i_s], as_rows.at[pl.ds(k * _SUB, _SUB)], sem))
            cps.append(pltpu.async_copy(ad_hbm.at[i_d], ad_rows.at[pl.ds(k * _SUB, _SUB)], sem))
        for cp in cps:
            cp.wait()
        for k in range(_NSUB):
            def add_body(e, c2, k=k):
                r = k * _SUB + e
                fr = r // 8
                sl = (r % 8) * 16
                gu_v[fr, pl.ds(sl, 16)] = s_rows[k, e, pl.ds(0, 16)] + d_rows[k, e, pl.ds(0, 16)]
                gf_v[fr, pl.ds(sl, 16)] = s_rows[k, e, pl.ds(16, 16)] + d_rows[k, e, pl.ds(16, 16)]
                ga_v[fr, pl.ds(sl, 16)] = (as_rows[pl.ds(r, 16)] + ad_rows[pl.ds(r, 16)])
                return c2
            lax.fori_loop(0, _SUB, add_body, 0)
        pltpu.sync_copy(gu_v, gu_hbm.at[pl.ds(base // 8, _CHUNK // 8)])
        pltpu.sync_copy(gf_v, gf_hbm.at[pl.ds(base // 8, _CHUNK // 8)])
        pltpu.sync_copy(ga_v, ga_hbm.at[pl.ds(base // 8, _CHUNK // 8)])
        return carry

    lax.fori_loop(0, n_chunks, chunk, 0)


def _mid_body(gu, gf, ga, emb, wue, wae, wedge, psel, bu, ba, be, upd, stats):
    i = pl.program_id(0)
    e = emb[...]
    u_e = jnp.dot(e, wue[...], preferred_element_type=jnp.float32) + bu[...]
    a_e = jnp.dot(e, wae[...], preferred_element_type=jnp.float32) + ba[...]
    ga_b = jnp.dot(ga[...], psel[...], preferred_element_type=jnp.float32)
    z = ga_b + a_e
    z = jnp.maximum(z, 0.2 * z)
    attn = 1.0 / (1.0 + jnp.exp(-z))
    upd_blk = (gu[...] + u_e) * attn
    upd[...] = upd_blk
    ef = jnp.dot(upd_blk, wedge[...], preferred_element_type=jnp.float32) + gf[...] + be[...]
    s = jnp.maximum(ef, 0.2 * ef)
    m_blk = jnp.max(s, axis=0, keepdims=True)

    @pl.when(i == 0)
    def _():
        stats[0:1, :] = m_blk
        stats[1:2, :] = jnp.sum(jnp.exp(s - m_blk), axis=0, keepdims=True)

    @pl.when(i > 0)
    def _():
        m_old = stats[0:1, :]
        l_old = stats[1:2, :]
        m_new = jnp.maximum(m_old, m_blk)
        stats[0:1, :] = m_new
        stats[1:2, :] = (l_old * jnp.exp(m_old - m_new)
                         + jnp.sum(jnp.exp(s - m_new), axis=0, keepdims=True))


def _final_body(upd, gf, wedge, be, stats, out, *, oc):
    u = upd[...]
    ef = jnp.dot(u, wedge[...], preferred_element_type=jnp.float32) + gf[...] + be[...]
    s = jnp.maximum(ef, 0.2 * ef)
    ms = stats[0:1, :]
    ls = stats[1:2, :]
    m_col = ms[:, 0:oc]
    for a in range(1, 8):
        m_col = jnp.maximum(m_col, ms[:, a * oc:(a + 1) * oc])
    l_col = jnp.zeros_like(m_col)
    for a in range(8):
        l_col = l_col + ls[:, a * oc:(a + 1) * oc] * jnp.exp(ms[:, a * oc:(a + 1) * oc] - m_col)
    m_t = jnp.concatenate([m_col] * 8, axis=1)
    l_t = jnp.concatenate([l_col] * 8, axis=1)
    out[...] = u * (jnp.exp(s - m_t) * (1.0 / l_t))


def kernel(edge_embeddings, edge_index, edge_attr, node_embeddings, num_nodes,
           W_attn, b_attn, W_update, b_update, W_edge, b_edge, W_node, b_node):
    E, ED = edge_embeddings.shape
    N, ND = node_embeddings.shape
    OC = W_edge.shape[1]
    f32 = jnp.float32

    src = edge_index[0].astype(jnp.int32)
    dst = edge_index[1].astype(jnp.int32)

    wS = jnp.concatenate([W_update[:ND], W_node], axis=1)
    wD = jnp.concatenate([W_update[ND:2 * ND], W_node], axis=1)
    wAs = W_attn[:ND]
    wAd = W_attn[ND:2 * ND]
    wue = W_update[2 * ND:]
    wae = W_attn[2 * ND:]
    bn_row = jnp.concatenate([jnp.zeros((OC,), f32), b_node]).reshape(1, 2 * OC)

    S, D, As, Ad = pl.pallas_call(
        _tables_body,
        out_shape=[
            jax.ShapeDtypeStruct((N, 2 * OC), f32),
            jax.ShapeDtypeStruct((N, 2 * OC), f32),
            jax.ShapeDtypeStruct((N, 1), f32),
            jax.ShapeDtypeStruct((N, 1), f32),
        ],
    )(node_embeddings, wS, wD, wAs, wAd, bn_row)

    edges_per_worker = E // _NW

    mesh = plsc.VectorSubcoreMesh(
        core_axis_name="c", subcore_axis_name="s",
        num_cores=_NC, num_subcores=_NS)
    sc_gather = pl.kernel(
        functools.partial(_sc_body, edges_per_worker=edges_per_worker),
        out_type=[
            jax.ShapeDtypeStruct((E // 8, 8 * OC), f32),
            jax.ShapeDtypeStruct((E // 8, 8 * OC), f32),
            jax.ShapeDtypeStruct((E // 8, 8 * OC), f32),
        ],
        mesh=mesh,
        scratch_types=[
            pltpu.VMEM((_CHUNK,), jnp.int32),
            pltpu.VMEM((_CHUNK,), jnp.int32),
            pltpu.VMEM((_NSUB, _SUB, 2 * OC), f32),
            pltpu.VMEM((_NSUB, _SUB, 2 * OC), f32),
            pltpu.VMEM((_CHUNK + 16,), f32),
            pltpu.VMEM((_CHUNK + 16,), f32),
            pltpu.VMEM((_CHUNK // 8, 8 * OC), f32),
            pltpu.VMEM((_CHUNK // 8, 8 * OC), f32),
            pltpu.VMEM((_CHUNK // 8, 8 * OC), f32),
            pltpu.SemaphoreType.DMA,
        ],
        compiler_params=pltpu.CompilerParams(use_tc_tiling_on_sc=False),
    )
    Gu, Gf, Ga = sc_gather(src, dst, S, D, As.reshape(N), Ad.reshape(N))
    emb_fold = edge_embeddings.reshape(E // 8, 8 * ED)

    eye8 = jnp.eye(8, dtype=f32)
    wue_k = jnp.kron(eye8, wue)
    wae_k = jnp.kron(eye8, wae @ jnp.ones((1, OC), f32))
    wedge_k = jnp.kron(eye8, W_edge)
    sel0 = jnp.zeros((OC, OC), f32).at[0, :].set(1.0)
    p_sel = jnp.kron(eye8, sel0)
    bu_t = jnp.tile(b_update.reshape(1, OC), (1, 8))
    ba_t = jnp.tile(b_attn.reshape(1, 1), (1, 8 * OC))
    be_t = jnp.tile(b_edge.reshape(1, OC), (1, 8))

    EF = E // 8
    nb = EF // _BF
    bspec_f = pl.BlockSpec((_BF, 8 * OC), lambda i: (i, 0))
    wspec = pl.BlockSpec((8 * OC, 8 * OC), lambda i: (0, 0))
    bias_spec = pl.BlockSpec((1, 8 * OC), lambda i: (0, 0))
    upd, stats = pl.pallas_call(
        _mid_body,
        grid=(nb,),
        in_specs=[bspec_f, bspec_f, bspec_f, bspec_f,
                  wspec, wspec, wspec, wspec, bias_spec, bias_spec, bias_spec],
        out_specs=[bspec_f, pl.BlockSpec((2, 8 * OC), lambda i: (0, 0))],
        out_shape=[
            jax.ShapeDtypeStruct((EF, 8 * OC), f32),
            jax.ShapeDtypeStruct((2, 8 * OC), f32),
        ],
    )(Gu, Gf, Ga, emb_fold, wue_k, wae_k, wedge_k, p_sel, bu_t, ba_t, be_t)

    refined_fold = pl.pallas_call(
        functools.partial(_final_body, oc=OC),
        grid=(nb,),
        in_specs=[bspec_f, bspec_f, wspec, bias_spec,
                  pl.BlockSpec((2, 8 * OC), lambda i: (0, 0))],
        out_specs=bspec_f,
        out_shape=jax.ShapeDtypeStruct((EF, 8 * OC), f32),
    )(upd, Gf, wedge_k, be_t, stats)

    return refined_fold.reshape(E, OC)


# speedup vs baseline: 4.1506x; 1.2262x over previous
import functools

import jax
import jax.numpy as jnp
from jax import lax
from jax.experimental import pallas as pl
from jax.experimental.pallas import tpu as pltpu
from jax.experimental.pallas import tpu_sc as plsc

_NC = 2
_NS = 16
_NW = _NC * _NS
_SUB = 80
_NSUB = 5
_CHUNK = _SUB * _NSUB

_BF = 2000


def _tables_body(node_ref, ws_ref, wd_ref, was_ref, wad_ref, bn_ref,
                 s_ref, d_ref, as_ref, ad_ref):
    x = node_ref[...]
    s_ref[...] = jnp.dot(x, ws_ref[...], preferred_element_type=jnp.float32) + bn_ref[...]
    d_ref[...] = jnp.dot(x, wd_ref[...], preferred_element_type=jnp.float32) + bn_ref[...]
    as_ref[...] = jnp.dot(x, was_ref[...], preferred_element_type=jnp.float32)
    ad_ref[...] = jnp.dot(x, wad_ref[...], preferred_element_type=jnp.float32)


def _sc_body(src_h, dst_h, s_hbm, d_hbm, as_hbm, ad_hbm,
             gu_hbm, gf_hbm, ga_hbm,
             idx_s, idx_d, s_rows, d_rows, as_rows, ad_rows,
             gu_v, gf_v, ga_v, sem_i0, sem_i1, sem_g0, sem_g1, sem_w0, sem_w1,
             *, edges_per_worker):
    sem_i = [sem_i0, sem_i1]
    sem_g = [sem_g0, sem_g1]
    sem_w = [sem_w0, sem_w1]
    cid = lax.axis_index("c")
    sid = lax.axis_index("s")
    wid = sid * _NC + cid
    wbase = wid * edges_per_worker
    n_chunks = edges_per_worker // _CHUNK
    zeros16 = jnp.zeros((16,), jnp.float32)
    for p in range(2):
        as_rows[p, pl.ds(_CHUNK, 16)] = zeros16
        ad_rows[p, pl.ds(_CHUNK, 16)] = zeros16

    def start_idx(c, p):
        pltpu.async_copy(src_h.at[pl.ds(wbase + c * _CHUNK, _CHUNK)], idx_s.at[p], sem_i[p])
        pltpu.async_copy(dst_h.at[pl.ds(wbase + c * _CHUNK, _CHUNK)], idx_d.at[p], sem_i[p])

    def wait_idx(p):
        pltpu.make_async_copy(src_h.at[pl.ds(wbase, _CHUNK)], idx_s.at[p], sem_i[p]).wait()
        pltpu.make_async_copy(dst_h.at[pl.ds(wbase, _CHUNK)], idx_d.at[p], sem_i[p]).wait()

    def fire_gathers(p):
        for k in range(_NSUB):
            i_s = idx_s.at[p, pl.ds(k * _SUB, _SUB)]
            i_d = idx_d.at[p, pl.ds(k * _SUB, _SUB)]
            pltpu.async_copy(s_hbm.at[i_s], s_rows.at[p, k], sem_g[p])
            pltpu.async_copy(d_hbm.at[i_d], d_rows.at[p, k], sem_g[p])
            pltpu.async_copy(as_hbm.at[i_s], as_rows.at[p, pl.ds(k * _SUB, _SUB)], sem_g[p])
            pltpu.async_copy(ad_hbm.at[i_d], ad_rows.at[p, pl.ds(k * _SUB, _SUB)], sem_g[p])

    def wait_gathers(p):
        for k in range(_NSUB):
            i_s = idx_s.at[p, pl.ds(k * _SUB, _SUB)]
            i_d = idx_d.at[p, pl.ds(k * _SUB, _SUB)]
            pltpu.make_async_copy(s_hbm.at[i_s], s_rows.at[p, k], sem_g[p]).wait()
            pltpu.make_async_copy(d_hbm.at[i_d], d_rows.at[p, k], sem_g[p]).wait()
            pltpu.make_async_copy(as_hbm.at[i_s], as_rows.at[p, pl.ds(k * _SUB, _SUB)], sem_g[p]).wait()
            pltpu.make_async_copy(ad_hbm.at[i_d], ad_rows.at[p, pl.ds(k * _SUB, _SUB)], sem_g[p]).wait()

    def compute(p):
        def qbody(q, c2):
            k = q // (_SUB // 8)
            e8 = (q % (_SUB // 8)) * 8
            for a in range(8):
                e = e8 + a
                gu_v[p, q, pl.ds(a * 16, 16)] = (
                    s_rows[p, k, e, pl.ds(0, 16)] + d_rows[p, k, e, pl.ds(0, 16)])
                gf_v[p, q, pl.ds(a * 16, 16)] = (
                    s_rows[p, k, e, pl.ds(16, 16)] + d_rows[p, k, e, pl.ds(16, 16)])
                r = q * 8 + a
                ga_v[p, q, pl.ds(a * 16, 16)] = (
                    as_rows[p, pl.ds(r, 16)] + ad_rows[p, pl.ds(r, 16)])
            return c2
        lax.fori_loop(0, _CHUNK // 8, qbody, 0)

    def fire_writes(c, p):
        fb = (wbase + c * _CHUNK) // 8
        pltpu.async_copy(gu_v.at[p], gu_hbm.at[pl.ds(fb, _CHUNK // 8)], sem_w[p])
        pltpu.async_copy(gf_v.at[p], gf_hbm.at[pl.ds(fb, _CHUNK // 8)], sem_w[p])
        pltpu.async_copy(ga_v.at[p], ga_hbm.at[pl.ds(fb, _CHUNK // 8)], sem_w[p])

    def wait_writes(p):
        fb = wbase // 8
        pltpu.make_async_copy(gu_v.at[p], gu_hbm.at[pl.ds(fb, _CHUNK // 8)], sem_w[p]).wait()
        pltpu.make_async_copy(gf_v.at[p], gf_hbm.at[pl.ds(fb, _CHUNK // 8)], sem_w[p]).wait()
        pltpu.make_async_copy(ga_v.at[p], ga_hbm.at[pl.ds(fb, _CHUNK // 8)], sem_w[p]).wait()

    def body(c, p, *, ww, ni, ng):
        if ng:
            wait_idx(1 - p)
            fire_gathers(1 - p)
        wait_gathers(p)
        if ww:
            wait_writes(p)
        if ni:
            start_idx(c + 2, p)
        compute(p)
        fire_writes(c, p)

    start_idx(0, 0)
    wait_idx(0)
    fire_gathers(0)
    start_idx(1, 1)
    body(0, 0, ww=False, ni=True, ng=True)
    body(1, 1, ww=False, ni=True, ng=True)

    def pair(t, carry):
        c0 = 2 + 2 * t
        body(c0, 0, ww=True, ni=True, ng=True)
        body(c0 + 1, 1, ww=True, ni=True, ng=True)
        return carry
    lax.fori_loop(0, (n_chunks - 5) // 2, pair, 0)

    body(n_chunks - 3, 0, ww=True, ni=True, ng=True)
    body(n_chunks - 2, 1, ww=True, ni=False, ng=True)
    body(n_chunks - 1, 0, ww=True, ni=False, ng=False)
    wait_writes(1)
    wait_writes(0)


def _mid_body(gu, gf, ga, emb, wue, wae, wedge, psel, bu, ba, be, upd, stats):
    i = pl.program_id(0)
    e = emb[...]
    u_e = jnp.dot(e, wue[...], preferred_element_type=jnp.float32) + bu[...]
    a_e = jnp.dot(e, wae[...], preferred_element_type=jnp.float32) + ba[...]
    ga_b = jnp.dot(ga[...], psel[...], preferred_element_type=jnp.float32)
    z = ga_b + a_e
    z = jnp.maximum(z, 0.2 * z)
    attn = 1.0 / (1.0 + jnp.exp(-z))
    upd_blk = (gu[...] + u_e) * attn
    upd[...] = upd_blk
    ef = jnp.dot(upd_blk, wedge[...], preferred_element_type=jnp.float32) + gf[...] + be[...]
    s = jnp.maximum(ef, 0.2 * ef)
    m_blk = jnp.max(s, axis=0, keepdims=True)

    @pl.when(i == 0)
    def _():
        stats[0:1, :] = m_blk
        stats[1:2, :] = jnp.sum(jnp.exp(s - m_blk), axis=0, keepdims=True)

    @pl.when(i > 0)
    def _():
        m_old = stats[0:1, :]
        l_old = stats[1:2, :]
        m_new = jnp.maximum(m_old, m_blk)
        stats[0:1, :] = m_new
        stats[1:2, :] = (l_old * jnp.exp(m_old - m_new)
                         + jnp.sum(jnp.exp(s - m_new), axis=0, keepdims=True))


def _final_body(upd, gf, wedge, be, stats, out, *, oc):
    u = upd[...]
    ef = jnp.dot(u, wedge[...], preferred_element_type=jnp.float32) + gf[...] + be[...]
    s = jnp.maximum(ef, 0.2 * ef)
    ms = stats[0:1, :]
    ls = stats[1:2, :]
    m_col = ms[:, 0:oc]
    for a in range(1, 8):
        m_col = jnp.maximum(m_col, ms[:, a * oc:(a + 1) * oc])
    l_col = jnp.zeros_like(m_col)
    for a in range(8):
        l_col = l_col + ls[:, a * oc:(a + 1) * oc] * jnp.exp(ms[:, a * oc:(a + 1) * oc] - m_col)
    m_t = jnp.concatenate([m_col] * 8, axis=1)
    l_t = jnp.concatenate([l_col] * 8, axis=1)
    out[...] = u * (jnp.exp(s - m_t) * (1.0 / l_t))


def kernel(edge_embeddings, edge_index, edge_attr, node_embeddings, num_nodes,
           W_attn, b_attn, W_update, b_update, W_edge, b_edge, W_node, b_node):
    E, ED = edge_embeddings.shape
    N, ND = node_embeddings.shape
    OC = W_edge.shape[1]
    f32 = jnp.float32

    src = edge_index[0].astype(jnp.int32)
    dst = edge_index[1].astype(jnp.int32)

    wS = jnp.concatenate([W_update[:ND], W_node], axis=1)
    wD = jnp.concatenate([W_update[ND:2 * ND], W_node], axis=1)
    wAs = W_attn[:ND]
    wAd = W_attn[ND:2 * ND]
    wue = W_update[2 * ND:]
    wae = W_attn[2 * ND:]
    bn_row = jnp.concatenate([jnp.zeros((OC,), f32), b_node]).reshape(1, 2 * OC)

    S, D, As, Ad = pl.pallas_call(
        _tables_body,
        out_shape=[
            jax.ShapeDtypeStruct((N, 2 * OC), f32),
            jax.ShapeDtypeStruct((N, 2 * OC), f32),
            jax.ShapeDtypeStruct((N, 1), f32),
            jax.ShapeDtypeStruct((N, 1), f32),
        ],
    )(node_embeddings, wS, wD, wAs, wAd, bn_row)

    edges_per_worker = E // _NW

    mesh = plsc.VectorSubcoreMesh(
        core_axis_name="c", subcore_axis_name="s",
        num_cores=_NC, num_subcores=_NS)
    sc_gather = pl.kernel(
        functools.partial(_sc_body, edges_per_worker=edges_per_worker),
        out_type=[
            jax.ShapeDtypeStruct((E // 8, 8 * OC), f32),
            jax.ShapeDtypeStruct((E // 8, 8 * OC), f32),
            jax.ShapeDtypeStruct((E // 8, 8 * OC), f32),
        ],
        mesh=mesh,
        scratch_types=[
            pltpu.VMEM((2, _CHUNK), jnp.int32),
            pltpu.VMEM((2, _CHUNK), jnp.int32),
            pltpu.VMEM((2, _NSUB, _SUB, 2 * OC), f32),
            pltpu.VMEM((2, _NSUB, _SUB, 2 * OC), f32),
            pltpu.VMEM((2, _CHUNK + 16), f32),
            pltpu.VMEM((2, _CHUNK + 16), f32),
            pltpu.VMEM((2, _CHUNK // 8, 8 * OC), f32),
            pltpu.VMEM((2, _CHUNK // 8, 8 * OC), f32),
            pltpu.VMEM((2, _CHUNK // 8, 8 * OC), f32),
            pltpu.SemaphoreType.DMA,
            pltpu.SemaphoreType.DMA,
            pltpu.SemaphoreType.DMA,
            pltpu.SemaphoreType.DMA,
            pltpu.SemaphoreType.DMA,
            pltpu.SemaphoreType.DMA,
        ],
        compiler_params=pltpu.CompilerParams(use_tc_tiling_on_sc=False),
    )
    Gu, Gf, Ga = sc_gather(src, dst, S, D, As.reshape(N), Ad.reshape(N))
    emb_fold = edge_embeddings.reshape(E // 8, 8 * ED)

    eye8 = jnp.eye(8, dtype=f32)
    wue_k = jnp.kron(eye8, wue)
    wae_k = jnp.kron(eye8, wae @ jnp.ones((1, OC), f32))
    wedge_k = jnp.kron(eye8, W_edge)
    sel0 = jnp.zeros((OC, OC), f32).at[0, :].set(1.0)
    p_sel = jnp.kron(eye8, sel0)
    bu_t = jnp.tile(b_update.reshape(1, OC), (1, 8))
    ba_t = jnp.tile(b_attn.reshape(1, 1), (1, 8 * OC))
    be_t = jnp.tile(b_edge.reshape(1, OC), (1, 8))

    EF = E // 8
    nb = EF // _BF
    bspec_f = pl.BlockSpec((_BF, 8 * OC), lambda i: (i, 0))
    wspec = pl.BlockSpec((8 * OC, 8 * OC), lambda i: (0, 0))
    bias_spec = pl.BlockSpec((1, 8 * OC), lambda i: (0, 0))
    upd, stats = pl.pallas_call(
        _mid_body,
        grid=(nb,),
        in_specs=[bspec_f, bspec_f, bspec_f, bspec_f,
                  wspec, wspec, wspec, wspec, bias_spec, bias_spec, bias_spec],
        out_specs=[bspec_f, pl.BlockSpec((2, 8 * OC), lambda i: (0, 0))],
        out_shape=[
            jax.ShapeDtypeStruct((EF, 8 * OC), f32),
            jax.ShapeDtypeStruct((2, 8 * OC), f32),
        ],
    )(Gu, Gf, Ga, emb_fold, wue_k, wae_k, wedge_k, p_sel, bu_t, ba_t, be_t)

    refined_fold = pl.pallas_call(
        functools.partial(_final_body, oc=OC),
        grid=(nb,),
        in_specs=[bspec_f, bspec_f, wspec, bias_spec,
                  pl.BlockSpec((2, 8 * OC), lambda i: (0, 0))],
        out_specs=bspec_f,
        out_shape=jax.ShapeDtypeStruct((EF, 8 * OC), f32),
    )(upd, Gf, wedge_k, be_t, stats)

    return refined_fold.reshape(E, OC)


# speedup vs baseline: 4.7443x; 1.1430x over previous
import functools

import jax
import jax.numpy as jnp
from jax import lax
from jax.experimental import pallas as pl
from jax.experimental.pallas import tpu as pltpu
from jax.experimental.pallas import tpu_sc as plsc

_NC = 2
_NS = 16
_NW = _NC * _NS
_SUB = 80
_NSUB = 5
_CHUNK = _SUB * _NSUB

_BF = 2000


def _tables_body(node_ref, ws_ref, wd_ref, waa_ref, bn_ref,
                 s_ref, d_ref, aa_ref):
    x = node_ref[...]
    s_ref[...] = jnp.dot(x, ws_ref[...], preferred_element_type=jnp.float32) + bn_ref[...]
    d_ref[...] = jnp.dot(x, wd_ref[...], preferred_element_type=jnp.float32) + bn_ref[...]
    aa_ref[...] = jnp.dot(x, waa_ref[...], preferred_element_type=jnp.float32)


def _sc_body(src_h, dst_h, s_hbm, d_hbm, as_hbm, ad_hbm,
             gu_hbm, gf_hbm, ga_hbm,
             idx_s, idx_d, s_rows, d_rows, as_rows, ad_rows,
             gu_v, gf_v, ga_v, sem_i0, sem_i1, sem_g0, sem_g1, sem_w0, sem_w1,
             *, edges_per_worker):
    sem_i = [sem_i0, sem_i1]
    sem_g = [sem_g0, sem_g1]
    sem_w = [sem_w0, sem_w1]
    cid = lax.axis_index("c")
    sid = lax.axis_index("s")
    wid = sid * _NC + cid
    wbase = wid * edges_per_worker
    n_chunks = edges_per_worker // _CHUNK
    zeros16 = jnp.zeros((16,), jnp.float32)
    for p in range(2):
        as_rows[p, pl.ds(_CHUNK, 16)] = zeros16
        ad_rows[p, pl.ds(_CHUNK, 16)] = zeros16

    def start_idx(c, p):
        pltpu.async_copy(src_h.at[pl.ds(wbase + c * _CHUNK, _CHUNK)], idx_s.at[p], sem_i[p])
        pltpu.async_copy(dst_h.at[pl.ds(wbase + c * _CHUNK, _CHUNK)], idx_d.at[p], sem_i[p])

    def wait_idx(p):
        pltpu.make_async_copy(src_h.at[pl.ds(wbase, _CHUNK)], idx_s.at[p], sem_i[p]).wait()
        pltpu.make_async_copy(dst_h.at[pl.ds(wbase, _CHUNK)], idx_d.at[p], sem_i[p]).wait()

    def fire_gathers(p):
        for k in range(_NSUB):
            i_s = idx_s.at[p, pl.ds(k * _SUB, _SUB)]
            i_d = idx_d.at[p, pl.ds(k * _SUB, _SUB)]
            pltpu.async_copy(s_hbm.at[i_s], s_rows.at[p, k], sem_g[p])
            pltpu.async_copy(d_hbm.at[i_d], d_rows.at[p, k], sem_g[p])
            pltpu.async_copy(as_hbm.at[i_s], as_rows.at[p, pl.ds(k * _SUB, _SUB)], sem_g[p])
            pltpu.async_copy(ad_hbm.at[i_d], ad_rows.at[p, pl.ds(k * _SUB, _SUB)], sem_g[p])

    def wait_gathers(p):
        for k in range(_NSUB):
            i_s = idx_s.at[p, pl.ds(k * _SUB, _SUB)]
            i_d = idx_d.at[p, pl.ds(k * _SUB, _SUB)]
            pltpu.make_async_copy(s_hbm.at[i_s], s_rows.at[p, k], sem_g[p]).wait()
            pltpu.make_async_copy(d_hbm.at[i_d], d_rows.at[p, k], sem_g[p]).wait()
            pltpu.make_async_copy(as_hbm.at[i_s], as_rows.at[p, pl.ds(k * _SUB, _SUB)], sem_g[p]).wait()
            pltpu.make_async_copy(ad_hbm.at[i_d], ad_rows.at[p, pl.ds(k * _SUB, _SUB)], sem_g[p]).wait()

    def compute(p):
        def qbody(q, c2):
            k = q // (_SUB // 8)
            e8 = (q % (_SUB // 8)) * 8
            for a in range(8):
                e = e8 + a
                gu_v[p, q, pl.ds(a * 16, 16)] = (
                    s_rows[p, k, e, pl.ds(0, 16)] + d_rows[p, k, e, pl.ds(0, 16)])
                gf_v[p, q, pl.ds(a * 16, 16)] = (
                    s_rows[p, k, e, pl.ds(16, 16)] + d_rows[p, k, e, pl.ds(16, 16)])
                r = q * 8 + a
                ga_v[p, q, pl.ds(a * 16, 16)] = (
                    as_rows[p, pl.ds(r, 16)] + ad_rows[p, pl.ds(r, 16)])
            return c2
        lax.fori_loop(0, _CHUNK // 8, qbody, 0)

    def fire_writes(c, p):
        fb = (wbase + c * _CHUNK) // 8
        pltpu.async_copy(gu_v.at[p], gu_hbm.at[pl.ds(fb, _CHUNK // 8)], sem_w[p])
        pltpu.async_copy(gf_v.at[p], gf_hbm.at[pl.ds(fb, _CHUNK // 8)], sem_w[p])
        pltpu.async_copy(ga_v.at[p], ga_hbm.at[pl.ds(fb, _CHUNK // 8)], sem_w[p])

    def wait_writes(p):
        fb = wbase // 8
        pltpu.make_async_copy(gu_v.at[p], gu_hbm.at[pl.ds(fb, _CHUNK // 8)], sem_w[p]).wait()
        pltpu.make_async_copy(gf_v.at[p], gf_hbm.at[pl.ds(fb, _CHUNK // 8)], sem_w[p]).wait()
        pltpu.make_async_copy(ga_v.at[p], ga_hbm.at[pl.ds(fb, _CHUNK // 8)], sem_w[p]).wait()

    def body(c, p, *, ww, ni, ng):
        if ng:
            wait_idx(1 - p)
            fire_gathers(1 - p)
        wait_gathers(p)
        if ww:
            wait_writes(p)
        if ni:
            start_idx(c + 2, p)
        compute(p)
        fire_writes(c, p)

    start_idx(0, 0)
    wait_idx(0)
    fire_gathers(0)
    start_idx(1, 1)
    body(0, 0, ww=False, ni=True, ng=True)
    body(1, 1, ww=False, ni=True, ng=True)

    def pair(t, carry):
        c0 = 2 + 2 * t
        body(c0, 0, ww=True, ni=True, ng=True)
        body(c0 + 1, 1, ww=True, ni=True, ng=True)
        return carry
    lax.fori_loop(0, (n_chunks - 5) // 2, pair, 0)

    body(n_chunks - 3, 0, ww=True, ni=True, ng=True)
    body(n_chunks - 2, 1, ww=True, ni=False, ng=True)
    body(n_chunks - 1, 0, ww=True, ni=False, ng=False)
    wait_writes(1)
    wait_writes(0)


def _mid_body(gu, gf, ga, emb, wue, wae, wedge, psel, bu, ba, be, upd, stats):
    i = pl.program_id(0)
    e = emb[...]
    u_e = jnp.dot(e, wue[...], preferred_element_type=jnp.float32) + bu[...]
    a_e = jnp.dot(e, wae[...], preferred_element_type=jnp.float32) + ba[...]
    ga_b = jnp.dot(ga[...], psel[...], preferred_element_type=jnp.float32)
    z = ga_b + a_e
    z = jnp.maximum(z, 0.2 * z)
    attn = 1.0 / (1.0 + jnp.exp(-z))
    upd_blk = (gu[...] + u_e) * attn
    upd[...] = upd_blk
    ef = jnp.dot(upd_blk, wedge[...], preferred_element_type=jnp.float32) + gf[...] + be[...]
    s = jnp.maximum(ef, 0.2 * ef)
    m_blk = jnp.max(s, axis=0, keepdims=True)

    @pl.when(i == 0)
    def _():
        stats[0:1, :] = m_blk
        stats[1:2, :] = jnp.sum(jnp.exp(s - m_blk), axis=0, keepdims=True)

    @pl.when(i > 0)
    def _():
        m_old = stats[0:1, :]
        l_old = stats[1:2, :]
        m_new = jnp.maximum(m_old, m_blk)
        stats[0:1, :] = m_new
        stats[1:2, :] = (l_old * jnp.exp(m_old - m_new)
                         + jnp.sum(jnp.exp(s - m_new), axis=0, keepdims=True))


def _final_body(upd, gf, wedge, be, stats, out, *, oc):
    u = upd[...]
    ef = jnp.dot(u, wedge[...], preferred_element_type=jnp.float32) + gf[...] + be[...]
    s = jnp.maximum(ef, 0.2 * ef)
    ms = stats[0:1, :]
    ls = stats[1:2, :]
    m_col = ms[:, 0:oc]
    for a in range(1, 8):
        m_col = jnp.maximum(m_col, ms[:, a * oc:(a + 1) * oc])
    l_col = jnp.zeros_like(m_col)
    for a in range(8):
        l_col = l_col + ls[:, a * oc:(a + 1) * oc] * jnp.exp(ms[:, a * oc:(a + 1) * oc] - m_col)
    m_t = jnp.concatenate([m_col] * 8, axis=1)
    l_t = jnp.concatenate([l_col] * 8, axis=1)
    out[...] = u * (jnp.exp(s - m_t) * (1.0 / l_t))


def kernel(edge_embeddings, edge_index, edge_attr, node_embeddings, num_nodes,
           W_attn, b_attn, W_update, b_update, W_edge, b_edge, W_node, b_node):
    E, ED = edge_embeddings.shape
    N, ND = node_embeddings.shape
    OC = W_edge.shape[1]
    f32 = jnp.float32

    src = edge_index[0].astype(jnp.int32)
    dst = edge_index[1].astype(jnp.int32)

    wS = jnp.concatenate([W_update[:ND], W_node], axis=1)
    wD = jnp.concatenate([W_update[ND:2 * ND], W_node], axis=1)
    wAs = W_attn[:ND]
    wAd = W_attn[ND:2 * ND]
    wue = W_update[2 * ND:]
    wae = W_attn[2 * ND:]
    bn_row = jnp.concatenate([jnp.zeros((OC,), f32), b_node]).reshape(1, 2 * OC)

    eye4 = jnp.eye(4, dtype=f32)
    wAA = jnp.concatenate([wAs, wAd, jnp.zeros((ND, 2 * OC - 2), f32)], axis=1)
    node_fold = node_embeddings.reshape(N // 4, 4 * ND)
    S4, D4, AA4 = pl.pallas_call(
        _tables_body,
        out_shape=[
            jax.ShapeDtypeStruct((N // 4, 8 * OC), f32),
            jax.ShapeDtypeStruct((N // 4, 8 * OC), f32),
            jax.ShapeDtypeStruct((N // 4, 8 * OC), f32),
        ],
    )(node_fold, jnp.kron(eye4, wS), jnp.kron(eye4, wD), jnp.kron(eye4, wAA),
      jnp.tile(bn_row, (1, 4)))
    S = S4.reshape(N, 2 * OC)
    D = D4.reshape(N, 2 * OC)
    AA = AA4.reshape(N, 2 * OC)
    As = AA[:, 0]
    Ad = AA[:, 1]

    edges_per_worker = E // _NW

    mesh = plsc.VectorSubcoreMesh(
        core_axis_name="c", subcore_axis_name="s",
        num_cores=_NC, num_subcores=_NS)
    sc_gather = pl.kernel(
        functools.partial(_sc_body, edges_per_worker=edges_per_worker),
        out_type=[
            jax.ShapeDtypeStruct((E // 8, 8 * OC), f32),
            jax.ShapeDtypeStruct((E // 8, 8 * OC), f32),
            jax.ShapeDtypeStruct((E // 8, 8 * OC), f32),
        ],
        mesh=mesh,
        scratch_types=[
            pltpu.VMEM((2, _CHUNK), jnp.int32),
            pltpu.VMEM((2, _CHUNK), jnp.int32),
            pltpu.VMEM((2, _NSUB, _SUB, 2 * OC), f32),
            pltpu.VMEM((2, _NSUB, _SUB, 2 * OC), f32),
            pltpu.VMEM((2, _CHUNK + 16), f32),
            pltpu.VMEM((2, _CHUNK + 16), f32),
            pltpu.VMEM((2, _CHUNK // 8, 8 * OC), f32),
            pltpu.VMEM((2, _CHUNK // 8, 8 * OC), f32),
            pltpu.VMEM((2, _CHUNK // 8, 8 * OC), f32),
            pltpu.SemaphoreType.DMA,
            pltpu.SemaphoreType.DMA,
            pltpu.SemaphoreType.DMA,
            pltpu.SemaphoreType.DMA,
            pltpu.SemaphoreType.DMA,
            pltpu.SemaphoreType.DMA,
        ],
        compiler_params=pltpu.CompilerParams(use_tc_tiling_on_sc=False),
    )
    Gu, Gf, Ga = sc_gather(src, dst, S, D, As, Ad)
    emb_fold = (edge_embeddings.T.reshape(ED, E // 8, 8)
                .transpose(1, 2, 0).reshape(E // 8, 8 * ED))

    eye8 = jnp.eye(8, dtype=f32)
    wue_k = jnp.kron(eye8, wue)
    wae_k = jnp.kron(eye8, wae @ jnp.ones((1, OC), f32))
    wedge_k = jnp.kron(eye8, W_edge)
    sel0 = jnp.zeros((OC, OC), f32).at[0, :].set(1.0)
    p_sel = jnp.kron(eye8, sel0)
    bu_t = jnp.tile(b_update.reshape(1, OC), (1, 8))
    ba_t = jnp.tile(b_attn.reshape(1, 1), (1, 8 * OC))
    be_t = jnp.tile(b_edge.reshape(1, OC), (1, 8))

    EF = E // 8
    nb = EF // _BF
    bspec_f = pl.BlockSpec((_BF, 8 * OC), lambda i: (i, 0))
    wspec = pl.BlockSpec((8 * OC, 8 * OC), lambda i: (0, 0))
    bias_spec = pl.BlockSpec((1, 8 * OC), lambda i: (0, 0))
    upd, stats = pl.pallas_call(
        _mid_body,
        grid=(nb,),
        in_specs=[bspec_f, bspec_f, bspec_f, bspec_f,
                  wspec, wspec, wspec, wspec, bias_spec, bias_spec, bias_spec],
        out_specs=[bspec_f, pl.BlockSpec((2, 8 * OC), lambda i: (0, 0))],
        out_shape=[
            jax.ShapeDtypeStruct((EF, 8 * OC), f32),
            jax.ShapeDtypeStruct((2, 8 * OC), f32),
        ],
    )(Gu, Gf, Ga, emb_fold, wue_k, wae_k, wedge_k, p_sel, bu_t, ba_t, be_t)

    refined_fold = pl.pallas_call(
        functools.partial(_final_body, oc=OC),
        grid=(nb,),
        in_specs=[bspec_f, bspec_f, wspec, bias_spec,
                  pl.BlockSpec((2, 8 * OC), lambda i: (0, 0))],
        out_specs=bspec_f,
        out_shape=jax.ShapeDtypeStruct((EF, 8 * OC), f32),
    )(upd, Gf, wedge_k, be_t, stats)

    return (refined_fold.reshape(EF, 8, OC)
            .transpose(2, 0, 1).reshape(OC, E).T)
